# baseline (speedup 1.0000x reference)
import functools

import jax
import jax.numpy as jnp
from jax import lax
from jax.experimental import pallas as pl
from jax.experimental.pallas import tpu as pltpu
from jax.experimental.pallas import tpu_sc as plsc

_COLS = 2048


def _argmin_block(e2_ref, z2_ref, iota_ref, emb_ref, zb_ref, idx_ref):
    emb = emb_ref[...]
    zb = zb_ref[...] * jnp.bfloat16(-2.0)
    dot_m2 = lax.dot_general(
        emb.astype(jnp.bfloat16), zb,
        (((1,), (1,)), ((), ())), preferred_element_type=jnp.float32,
    )
    ncodes, cols = dot_m2.shape
    e2 = e2_ref[...]
    iota = iota_ref[...]
    z2b = jnp.broadcast_to(z2_ref[...], (8, cols))
    val = jnp.full((8, cols), jnp.inf, jnp.float32)
    idxv = jnp.zeros((8, cols), jnp.float32)
    for k in range(ncodes // 8):
        sl = slice(8 * k, 8 * k + 8)
        distk = (z2b + e2[sl]) + dot_m2[sl, :]
        idxv = jnp.where(distk < val, jnp.broadcast_to(iota[sl], (8, cols)),
                         idxv)
        val = jnp.minimum(val, distk)
    m = jnp.min(val, axis=0, keepdims=True)
    idx_f = jnp.min(jnp.where(val == m, idxv, float(ncodes)),
                    axis=0, keepdims=True)
    idx = idx_f.astype(jnp.int32)
    idx_ref[...] = idx.reshape(idx.shape[1] // 128, 128)


def _compute_indices(z, embedding):
    n, d = z.shape
    ncodes = embedding.shape[0]
    z2 = jnp.sum(z ** 2, axis=1)[None, :]
    e2 = jnp.sum(embedding ** 2, axis=1)[:, None]
    iota_col = jnp.arange(ncodes, dtype=jnp.float32)[:, None]
    zb = z.astype(jnp.bfloat16)
    rows_per_blk = _COLS // 128
    idx2 = pl.pallas_call(
        _argmin_block,
        grid=(n // _COLS,),
        in_specs=[
            pl.BlockSpec((ncodes, 1), lambda i: (0, 0)),
            pl.BlockSpec((1, _COLS), lambda i: (0, i)),
            pl.BlockSpec((ncodes, 1), lambda i: (0, 0)),
            pl.BlockSpec((ncodes, d), lambda i: (0, 0)),
            pl.BlockSpec((_COLS, d), lambda i: (i, 0)),
        ],
        out_specs=pl.BlockSpec((rows_per_blk, 128), lambda i: (i, 0)),
        out_shape=jax.ShapeDtypeStruct((n // 128, 128), jnp.int32),
    )(e2, z2, iota_col, embedding, zb)
    return idx2.reshape(n)


@functools.lru_cache(maxsize=None)
def _make_gather(n, d, b_per_w):
    mesh = plsc.VectorSubcoreMesh(core_axis_name="c", subcore_axis_name="s")

    @functools.partial(
        pl.kernel, mesh=mesh,
        out_type=jax.ShapeDtypeStruct((n, d), jnp.float32),
        compiler_params=pltpu.CompilerParams(use_tc_tiling_on_sc=False),
        scratch_types=[
            pltpu.VMEM((b_per_w,), jnp.int32),
            pltpu.VMEM((b_per_w, d), jnp.float32),
            pltpu.SemaphoreType.DMA,
        ],
    )
    def gather(table_hbm, idx_hbm, out_hbm, idx_v, rows_v, sem):
        wid = lax.axis_index("s") * 2 + lax.axis_index("c")
        base = wid * b_per_w
        pltpu.sync_copy(idx_hbm.at[pl.ds(base, b_per_w)], idx_v)
        pltpu.async_copy(table_hbm.at[idx_v], rows_v, sem).wait()
        pltpu.sync_copy(rows_v, out_hbm.at[pl.ds(base, b_per_w)])

    return gather


def kernel(z, embedding):
    n, d = z.shape
    table = embedding.astype(jnp.bfloat16).astype(jnp.float32)
    half = n // 2
    gather = _make_gather(half, d, half // 32)
    outs = []
    for h in range(2):
        idx_h = _compute_indices(z[h * half:(h + 1) * half], embedding)
        outs.append(gather(table, idx_h))
    return jnp.concatenate(outs, axis=0)


# speedup vs baseline: 1.2638x; 1.2638x over previous
import functools

import jax
import jax.numpy as jnp
from jax import lax
from jax.experimental import pallas as pl
from jax.experimental.pallas import tpu as pltpu
from jax.experimental.pallas import tpu_sc as plsc

_COLS = 2048


def _argmin_block(e2_ref, z2_ref, iota_ref, emb_ref, zb_ref, idx_ref):
    emb = emb_ref[...]
    zb = zb_ref[...] * jnp.bfloat16(-2.0)
    dot_m2 = lax.dot_general(
        emb.astype(jnp.bfloat16), zb,
        (((1,), (1,)), ((), ())), preferred_element_type=jnp.float32,
    )
    ncodes, cols = dot_m2.shape
    e2 = e2_ref[...]
    iota = iota_ref[...]
    z2b = jnp.broadcast_to(z2_ref[...], (8, cols))
    val = jnp.full((8, cols), jnp.inf, jnp.float32)
    idxv = jnp.zeros((8, cols), jnp.float32)
    for k in range(ncodes // 8):
        sl = slice(8 * k, 8 * k + 8)
        distk = (z2b + e2[sl]) + dot_m2[sl, :]
        idxv = jnp.where(distk < val, jnp.broadcast_to(iota[sl], (8, cols)),
                         idxv)
        val = jnp.minimum(val, distk)
    m = jnp.min(val, axis=0, keepdims=True)
    idx_f = jnp.min(jnp.where(val == m, idxv, float(ncodes)),
                    axis=0, keepdims=True)
    idx = idx_f.astype(jnp.int32)
    idx_ref[...] = idx.reshape(idx.shape[1] // 128, 128)


def _compute_indices(z, embedding):
    n, d = z.shape
    ncodes = embedding.shape[0]
    z2 = jnp.sum(z ** 2, axis=1)[None, :]
    e2 = jnp.sum(embedding ** 2, axis=1)[:, None]
    iota_col = jnp.arange(ncodes, dtype=jnp.float32)[:, None]
    zb = z.astype(jnp.bfloat16)
    rows_per_blk = _COLS // 128
    idx2 = pl.pallas_call(
        _argmin_block,
        grid=(n // _COLS,),
        in_specs=[
            pl.BlockSpec((ncodes, 1), lambda i: (0, 0)),
            pl.BlockSpec((1, _COLS), lambda i: (0, i)),
            pl.BlockSpec((ncodes, 1), lambda i: (0, 0)),
            pl.BlockSpec((ncodes, d), lambda i: (0, 0)),
            pl.BlockSpec((_COLS, d), lambda i: (i, 0)),
        ],
        out_specs=pl.BlockSpec((rows_per_blk, 128), lambda i: (i, 0)),
        out_shape=jax.ShapeDtypeStruct((n // 128, 128), jnp.int32),
    )(e2, z2, iota_col, embedding, zb)
    return idx2.reshape(n)


@functools.lru_cache(maxsize=None)
def _make_gather(n, d, b_per_w):
    mesh = plsc.VectorSubcoreMesh(core_axis_name="c", subcore_axis_name="s")

    @functools.partial(
        pl.kernel, mesh=mesh,
        out_type=jax.ShapeDtypeStruct((n, d), jnp.float32),
        compiler_params=pltpu.CompilerParams(use_tc_tiling_on_sc=False),
        scratch_types=[
            pltpu.VMEM((b_per_w,), jnp.int32),
            pltpu.VMEM((b_per_w, d), jnp.float32),
            pltpu.SemaphoreType.DMA,
        ],
    )
    def gather(table_hbm, idx_hbm, out_hbm, idx_v, rows_v, sem):
        wid = lax.axis_index("s") * 2 + lax.axis_index("c")
        base = wid * b_per_w
        pltpu.sync_copy(idx_hbm.at[pl.ds(base, b_per_w)], idx_v)
        pltpu.async_copy(table_hbm.at[idx_v], rows_v, sem).wait()
        pltpu.sync_copy(rows_v, out_hbm.at[pl.ds(base, b_per_w)])

    return gather


def kernel(z, embedding):
    n, d = z.shape
    idx = _compute_indices(z, embedding)
    table = embedding.astype(jnp.bfloat16).astype(jnp.float32)
    return _make_gather(n, d, n // 32)(table, idx)


# speedup vs baseline: 1.2885x; 1.0195x over previous
import functools

import jax
import jax.numpy as jnp
from jax import lax
from jax.experimental import pallas as pl
from jax.experimental.pallas import tpu as pltpu
from jax.experimental.pallas import tpu_sc as plsc

_COLS = 4096


def _argmin_block(e2_ref, z2_ref, iota_ref, emb_ref, zb_ref, idx_ref):
    emb = emb_ref[...]
    zb = zb_ref[...] * jnp.bfloat16(-2.0)
    dot_m2 = lax.dot_general(
        emb.astype(jnp.bfloat16), zb,
        (((1,), (1,)), ((), ())), preferred_element_type=jnp.float32,
    )
    ncodes, cols = dot_m2.shape
    e2 = e2_ref[...]
    iota = iota_ref[...]
    z2b = jnp.broadcast_to(z2_ref[...], (8, cols))
    val = jnp.full((8, cols), jnp.inf, jnp.float32)
    idxv = jnp.zeros((8, cols), jnp.float32)
    for k in range(ncodes // 8):
        sl = slice(8 * k, 8 * k + 8)
        distk = (z2b + e2[sl]) + dot_m2[sl, :]
        idxv = jnp.where(distk < val, jnp.broadcast_to(iota[sl], (8, cols)),
                         idxv)
        val = jnp.minimum(val, distk)
    m = jnp.min(val, axis=0, keepdims=True)
    idx_f = jnp.min(jnp.where(val == m, idxv, float(ncodes)),
                    axis=0, keepdims=True)
    idx = idx_f.astype(jnp.int32)
    idx_ref[...] = idx.reshape(idx.shape[1] // 128, 128)


def _compute_indices(z, embedding):
    n, d = z.shape
    ncodes = embedding.shape[0]
    z2 = jnp.sum(z ** 2, axis=1)[None, :]
    e2 = jnp.sum(embedding ** 2, axis=1)[:, None]
    iota_col = jnp.arange(ncodes, dtype=jnp.float32)[:, None]
    zb = z.astype(jnp.bfloat16)
    rows_per_blk = _COLS // 128
    idx2 = pl.pallas_call(
        _argmin_block,
        grid=(n // _COLS,),
        in_specs=[
            pl.BlockSpec((ncodes, 1), lambda i: (0, 0)),
            pl.BlockSpec((1, _COLS), lambda i: (0, i)),
            pl.BlockSpec((ncodes, 1), lambda i: (0, 0)),
            pl.BlockSpec((ncodes, d), lambda i: (0, 0)),
            pl.BlockSpec((_COLS, d), lambda i: (i, 0)),
        ],
        out_specs=pl.BlockSpec((rows_per_blk, 128), lambda i: (i, 0)),
        out_shape=jax.ShapeDtypeStruct((n // 128, 128), jnp.int32),
    )(e2, z2, iota_col, embedding, zb)
    return idx2.reshape(n)


@functools.lru_cache(maxsize=None)
def _make_gather(n, d, b_per_w):
    mesh = plsc.VectorSubcoreMesh(core_axis_name="c", subcore_axis_name="s")

    @functools.partial(
        pl.kernel, mesh=mesh,
        out_type=jax.ShapeDtypeStruct((n, d), jnp.float32),
        compiler_params=pltpu.CompilerParams(use_tc_tiling_on_sc=False),
        scratch_types=[
            pltpu.VMEM((b_per_w,), jnp.int32),
            pltpu.VMEM((b_per_w, d), jnp.float32),
            pltpu.SemaphoreType.DMA,
        ],
    )
    def gather(table_hbm, idx_hbm, out_hbm, idx_v, rows_v, sem):
        wid = lax.axis_index("s") * 2 + lax.axis_index("c")
        base = wid * b_per_w
        pltpu.sync_copy(idx_hbm.at[pl.ds(base, b_per_w)], idx_v)
        pltpu.async_copy(table_hbm.at[idx_v], rows_v, sem).wait()
        pltpu.sync_copy(rows_v, out_hbm.at[pl.ds(base, b_per_w)])

    return gather


def kernel(z, embedding):
    n, d = z.shape
    idx = _compute_indices(z, embedding)
    table = embedding.astype(jnp.bfloat16).astype(jnp.float32)
    return _make_gather(n, d, n // 32)(table, idx)


# speedup vs baseline: 1.2929x; 1.0034x over previous
import functools

import jax
import jax.numpy as jnp
from jax import lax
from jax.experimental import pallas as pl
from jax.experimental.pallas import tpu as pltpu
from jax.experimental.pallas import tpu_sc as plsc

_COLS = 8192


def _argmin_block(e2_ref, z2_ref, iota_ref, emb_ref, zb_ref, idx_ref):
    emb = emb_ref[...]
    zb = zb_ref[...] * jnp.bfloat16(-2.0)
    dot_m2 = lax.dot_general(
        emb.astype(jnp.bfloat16), zb,
        (((1,), (1,)), ((), ())), preferred_element_type=jnp.float32,
    )
    ncodes, cols = dot_m2.shape
    e2 = e2_ref[...]
    iota = iota_ref[...]
    z2b = jnp.broadcast_to(z2_ref[...], (8, cols))
    val = jnp.full((8, cols), jnp.inf, jnp.float32)
    idxv = jnp.zeros((8, cols), jnp.float32)
    for k in range(ncodes // 8):
        sl = slice(8 * k, 8 * k + 8)
        distk = (z2b + e2[sl]) + dot_m2[sl, :]
        idxv = jnp.where(distk < val, jnp.broadcast_to(iota[sl], (8, cols)),
                         idxv)
        val = jnp.minimum(val, distk)
    m = jnp.min(val, axis=0, keepdims=True)
    idx_f = jnp.min(jnp.where(val == m, idxv, float(ncodes)),
                    axis=0, keepdims=True)
    idx = idx_f.astype(jnp.int32)
    idx_ref[...] = idx.reshape(idx.shape[1] // 128, 128)


def _compute_indices(z, embedding):
    n, d = z.shape
    ncodes = embedding.shape[0]
    z2 = jnp.sum(z ** 2, axis=1)[None, :]
    e2 = jnp.sum(embedding ** 2, axis=1)[:, None]
    iota_col = jnp.arange(ncodes, dtype=jnp.float32)[:, None]
    zb = z.astype(jnp.bfloat16)
    rows_per_blk = _COLS // 128
    idx2 = pl.pallas_call(
        _argmin_block,
        grid=(n // _COLS,),
        in_specs=[
            pl.BlockSpec((ncodes, 1), lambda i: (0, 0)),
            pl.BlockSpec((1, _COLS), lambda i: (0, i)),
            pl.BlockSpec((ncodes, 1), lambda i: (0, 0)),
            pl.BlockSpec((ncodes, d), lambda i: (0, 0)),
            pl.BlockSpec((_COLS, d), lambda i: (i, 0)),
        ],
        out_specs=pl.BlockSpec((rows_per_blk, 128), lambda i: (i, 0)),
        out_shape=jax.ShapeDtypeStruct((n // 128, 128), jnp.int32),
    )(e2, z2, iota_col, embedding, zb)
    return idx2.reshape(n)


@functools.lru_cache(maxsize=None)
def _make_gather(n, d, b_per_w):
    mesh = plsc.VectorSubcoreMesh(core_axis_name="c", subcore_axis_name="s")

    @functools.partial(
        pl.kernel, mesh=mesh,
        out_type=jax.ShapeDtypeStruct((n, d), jnp.float32),
        compiler_params=pltpu.CompilerParams(use_tc_tiling_on_sc=False),
        scratch_types=[
            pltpu.VMEM((b_per_w,), jnp.int32),
            pltpu.VMEM((b_per_w, d), jnp.float32),
            pltpu.SemaphoreType.DMA,
        ],
    )
    def gather(table_hbm, idx_hbm, out_hbm, idx_v, rows_v, sem):
        wid = lax.axis_index("s") * 2 + lax.axis_index("c")
        base = wid * b_per_w
        pltpu.sync_copy(idx_hbm.at[pl.ds(base, b_per_w)], idx_v)
        pltpu.async_copy(table_hbm.at[idx_v], rows_v, sem).wait()
        pltpu.sync_copy(rows_v, out_hbm.at[pl.ds(base, b_per_w)])

    return gather


def kernel(z, embedding):
    n, d = z.shape
    idx = _compute_indices(z, embedding)
    table = embedding.astype(jnp.bfloat16).astype(jnp.float32)
    return _make_gather(n, d, n // 32)(table, idx)
